# baseline (speedup 1.0000x reference)
import functools

import jax
import jax.numpy as jnp
from jax import lax
from jax.experimental import pallas as pl
from jax.experimental.pallas import tpu as pltpu

B, T0, CIN = 32, 2048, 64
T = 512
C1 = 256
D = 64
V = 1024
NTOT = B * T


def _shd(a):
    return jnp.concatenate([jnp.zeros_like(a[:1]), a[:-1]], axis=0)


def _shu(a):
    return jnp.concatenate([a[1:], jnp.zeros_like(a[:1])], axis=0)


def _body(xe_ref, xo_ref, w1_ref, w2_ref, w3_ref,
          wr10_ref, wr20_ref, wr11_ref, wr21_ref, wq_ref,
          et_ref, emb_ref,
          b1_ref, b2_ref, b3_ref, br10_ref, br20_ref, br11_ref, br21_ref,
          bq_ref,
          ze_ref, zq_ref, enc_ref, idx_ref, loss_ref, perp_ref,
          hist_s):
    b = pl.program_id(0)
    nb = pl.num_programs(0)

    w1 = w1_ref[...]
    y1e = jax.nn.relu(jnp.dot(xe_ref[0], w1) + b1_ref[...])
    y1o = jax.nn.relu(jnp.dot(xo_ref[0], w1) + b1_ref[...])

    w2 = w2_ref[...]
    y2 = (jnp.dot(_shd(y1o), w2[0:256]) + jnp.dot(y1e, w2[256:512])
          + jnp.dot(y1o, w2[512:768]) + jnp.dot(_shu(y1e), w2[768:1024])
          + b2_ref[...])
    a2 = jax.nn.relu(y2)

    w3 = w3_ref[...]
    h = (jnp.dot(_shd(a2), w3[0:256]) + jnp.dot(a2, w3[256:512])
         + jnp.dot(_shu(a2), w3[512:768]) + b3_ref[...])

    for wr1_ref, br1_ref, wr2_ref, br2_ref in (
            (wr10_ref, br10_ref, wr20_ref, br20_ref),
            (wr11_ref, br11_ref, wr21_ref, br21_ref)):
        a = jax.nn.relu(h)
        wr1 = wr1_ref[...]
        r1 = (jnp.dot(_shd(a), wr1[0:64]) + jnp.dot(a, wr1[64:128])
              + jnp.dot(_shu(a), wr1[128:192]) + br1_ref[...])
        r2 = jnp.dot(jax.nn.relu(r1), wr2_ref[...]) + br2_ref[...]
        h = h + r2

    z = jnp.dot(jax.nn.relu(h), wq_ref[...]) + bq_ref[...]
    ze_ref[0] = z

    et = et_ref[...]
    z2 = jnp.sum(z * z, axis=1, keepdims=True)
    e2 = jnp.sum(et * et, axis=0, keepdims=True)
    d2 = (z2 + e2) - 2.0 * jnp.dot(z, et)
    m = jnp.min(d2, axis=1, keepdims=True)
    li = lax.broadcasted_iota(jnp.int32, (T, V), 1)
    idx = jnp.min(jnp.where(d2 == m, li, V), axis=1, keepdims=True)
    idx_ref[0] = idx
    enc = (li == idx).astype(jnp.float32)
    enc_ref[0] = enc
    zq_ref[0] = jnp.dot(enc, emb_ref[...])

    lsum = jnp.sum(m, keepdims=True).reshape(1, 1)
    hvec = jnp.sum(enc, axis=0, keepdims=True)
    lacc = jnp.where(b == 0, lsum, loss_ref[...] + lsum)
    hacc = jnp.where(b == 0, hvec, hist_s[...] + hvec)
    hist_s[...] = hacc
    last = b == nb - 1
    loss_ref[...] = jnp.where(
        last, 1.25 * lacc / jnp.float32(NTOT * D), lacc)
    p = hacc / jnp.float32(NTOT)
    ent = jnp.sum(p * jnp.log(p + 1e-10), keepdims=True).reshape(1, 1)
    perp_ref[...] = jnp.where(last, jnp.exp(-ent), jnp.zeros_like(ent))


def kernel(x, W1, b1, W2, b2, W3, b3, Wr1_0, br1_0, Wr2_0, br2_0,
           Wr1_1, br1_1, Wr2_1, br2_1, Wq, bq, embedding):
    f32 = jnp.float32
    xt = jnp.transpose(x, (0, 2, 1))
    xp = jnp.pad(xt, ((0, 0), (1, 1), (0, 0)))
    xe = xp[:, 0:2048].reshape(B, T, 4 * CIN)
    xo = xp[:, 2:2050].reshape(B, T, 4 * CIN)

    w1f = W1.transpose(2, 1, 0).reshape(4 * CIN, C1)
    w2f = W2.transpose(2, 1, 0).reshape(4 * C1, C1)
    w3f = W3.transpose(2, 1, 0).reshape(3 * C1, D)
    wr10 = Wr1_0.transpose(2, 1, 0).reshape(3 * D, D)
    wr11 = Wr1_1.transpose(2, 1, 0).reshape(3 * D, D)
    wr20 = Wr2_0[:, :, 0].T
    wr21 = Wr2_1[:, :, 0].T
    wqf = Wq[:, :, 0].T
    et = embedding.T

    row = lambda v: v.reshape(1, -1)
    full = lambda s: pl.BlockSpec(s, lambda b: (0,) * len(s))

    out_shapes = (
        jax.ShapeDtypeStruct((B, T, D), f32),
        jax.ShapeDtypeStruct((B, T, D), f32),
        jax.ShapeDtypeStruct((B, T, V), f32),
        jax.ShapeDtypeStruct((B, T, 1), jnp.int32),
        jax.ShapeDtypeStruct((1, 1), f32),
        jax.ShapeDtypeStruct((1, 1), f32),
    )
    out_specs = (
        pl.BlockSpec((1, T, D), lambda b: (b, 0, 0)),
        pl.BlockSpec((1, T, D), lambda b: (b, 0, 0)),
        pl.BlockSpec((1, T, V), lambda b: (b, 0, 0)),
        pl.BlockSpec((1, T, 1), lambda b: (b, 0, 0)),
        full((1, 1)),
        full((1, 1)),
    )
    in_specs = [
        pl.BlockSpec((1, T, 4 * CIN), lambda b: (b, 0, 0)),
        pl.BlockSpec((1, T, 4 * CIN), lambda b: (b, 0, 0)),
        full((4 * CIN, C1)), full((4 * C1, C1)), full((3 * C1, D)),
        full((3 * D, D)), full((D, D)), full((3 * D, D)), full((D, D)),
        full((D, D)), full((D, V)), full((V, D)),
        full((1, C1)), full((1, C1)), full((1, D)), full((1, D)),
        full((1, D)), full((1, D)), full((1, D)), full((1, D)),
    ]

    ze, zq, enc, idx3, loss2, perp2 = pl.pallas_call(
        _body,
        grid=(B,),
        in_specs=in_specs,
        out_specs=out_specs,
        out_shape=out_shapes,
        scratch_shapes=[pltpu.VMEM((1, V), f32)],
    )(xe, xo, w1f, w2f, w3f, wr10, wr20, wr11, wr21, wqf, et, embedding,
      row(b1), row(b2), row(b3), row(br1_0), row(br2_0), row(br1_1),
      row(br2_1), row(bq))

    loss = loss2.reshape(())
    perplexity = perp2.reshape(())
    idx = idx3.reshape(NTOT)
    enc2 = enc.reshape(NTOT, V)
    z_q_st = zq
    return (loss, z_q_st, perplexity, ze, enc2, idx)
